# baseline (speedup 1.0000x reference)
import jax
import jax.numpy as jnp
from jax import lax
from jax.experimental import pallas as pl
from jax.experimental.pallas import tpu as pltpu
from jax.experimental.pallas import tpu_sc as plsc

EMBED_DIM = 16
SEQ_LEN = 200
BATCH = 4096

NC = 2
NS = 16
NW = NC * NS

ST = SEQ_LEN // 8
BT = BATCH // 128
N_BLOCKS = ST * BT
BLK_PER_W = N_BLOCKS // NW
ROWS_BLK = 8 * 128
OB_LEN = ROWS_BLK * EMBED_DIM


def _positional_encoding_host(seq_len, embed_dim):
    pos = jnp.arange(seq_len, dtype=jnp.float32)[:, None]
    dim = jnp.arange(embed_dim, dtype=jnp.float32)[None, :]
    angle = pos / jnp.power(10000.0, 2.0 * dim / float(embed_dim))
    is_even = (jnp.arange(embed_dim)[None, :] % 2) == 0
    return jnp.where(is_even, jnp.sin(angle), jnp.cos(angle))


def _sc_body(table_hbm, xr_hbm, pe_hbm, out_hbm,
             idx_all, rows0, rows1, ob0, ob1, pe_v,
             gsem0, gsem1, osem0, osem1):
    w = lax.axis_index("s") * NC + lax.axis_index("c")
    rows_b = (rows0, rows1)
    ob_b = (ob0, ob1)
    gsem_b = (gsem0, gsem1)
    osem_b = (osem0, osem1)

    pltpu.sync_copy(xr_hbm.at[pl.ds(w * BLK_PER_W, BLK_PER_W)], idx_all)
    pltpu.sync_copy(pe_hbm, pe_v)
    lane = lax.iota(jnp.int32, 16)
    perm = (lane // 8) * 1024 + (lane % 8) * 128

    def tiles(k):
        blk = w * BLK_PER_W + k
        return blk // BT, blk % BT

    def gather_start(k, b):
        pltpu.async_copy(table_hbm.at[idx_all.at[k]], rows_b[b], gsem_b[b])

    def gather_wait(k, b):
        pltpu.make_async_copy(table_hbm.at[idx_all.at[k]], rows_b[b],
                              gsem_b[b]).wait()

    def store_start(k, b):
        st, bt = tiles(k)
        obuf = ob_b[b]
        for sr in range(8):
            for dt in range(2):
                pltpu.async_copy(
                    obuf.at[pl.ds(sr * 2048 + dt * 1024, 1024)],
                    out_hbm.at[st * 8 + sr, dt, bt], osem_b[b])

    def store_wait(k, b):
        st, bt = tiles(k)
        obuf = ob_b[b]
        for sr in range(8):
            for dt in range(2):
                pltpu.make_async_copy(
                    obuf.at[pl.ds(sr * 2048 + dt * 1024, 1024)],
                    out_hbm.at[st * 8 + sr, dt, bt], osem_b[b]).wait()

    def transpose(k, b):
        st, _ = tiles(k)
        rows_v = rows_b[b]
        obuf = ob_b[b]

        def sr_step(sr, _):
            pe_row = pe_v[st * 8 + sr]
            base0 = sr * 2048
            for bl in range(128):
                vals = rows_v[sr * 128 + bl] + pe_row
                plsc.store_scatter(obuf, [perm + (base0 + bl)], vals)
            return 0

        lax.fori_loop(0, 8, sr_step, 0)

    gather_start(0, 0)

    def pair_step(kk, carry):
        for par in range(2):
            k = 2 * kk + par
            b = par
            gather_wait(k, b)
            gather_start(k + 1, 1 - b)

            @pl.when(k >= 2)
            def _():
                store_wait(k - 2, b)

            transpose(k, b)
            store_start(k, b)
        return carry

    lax.fori_loop(0, (BLK_PER_W - 1) // 2, pair_step, 0)

    last = BLK_PER_W - 1
    gather_wait(last, 0)
    store_wait(last - 2, 0)
    transpose(last, 0)
    store_start(last, 0)
    store_wait(last - 1, 1)
    store_wait(last, 0)


@jax.jit
def _embed_lookup(xr, table, pe):
    mesh = plsc.VectorSubcoreMesh(core_axis_name="c", subcore_axis_name="s")
    return pl.kernel(
        _sc_body,
        out_type=jax.ShapeDtypeStruct((SEQ_LEN, 2, BT, ROWS_BLK),
                                      jnp.float32),
        mesh=mesh,
        scratch_types=[
            pltpu.VMEM((N_BLOCKS // NW, ROWS_BLK), jnp.int32),
            pltpu.VMEM((ROWS_BLK, EMBED_DIM), jnp.float32),
            pltpu.VMEM((ROWS_BLK, EMBED_DIM), jnp.float32),
            pltpu.VMEM((OB_LEN,), jnp.float32),
            pltpu.VMEM((OB_LEN,), jnp.float32),
            pltpu.VMEM((SEQ_LEN, EMBED_DIM), jnp.float32),
            pltpu.SemaphoreType.DMA,
            pltpu.SemaphoreType.DMA,
            pltpu.SemaphoreType.DMA,
            pltpu.SemaphoreType.DMA,
        ],
        compiler_params=pltpu.CompilerParams(use_tc_tiling_on_sc=False,
                                             needs_layout_passes=False),
    )(table, xr, pe)


def kernel(x, table):
    pe = _positional_encoding_host(SEQ_LEN, EMBED_DIM)
    xr = (x.astype(jnp.int32).T
          .reshape(ST, 8, BT, 128)
          .transpose(0, 2, 1, 3)
          .reshape(N_BLOCKS, ROWS_BLK))
    out5 = _embed_lookup(xr, table, pe)
    return (out5.reshape(SEQ_LEN, 2, BT, 8, 128)
            .transpose(2, 4, 0, 1, 3)
            .reshape(BATCH, SEQ_LEN, EMBED_DIM))


# speedup vs baseline: 1.2148x; 1.2148x over previous
import jax
import jax.numpy as jnp
from jax import lax
from jax.experimental import pallas as pl
from jax.experimental.pallas import tpu as pltpu
from jax.experimental.pallas import tpu_sc as plsc

EMBED_DIM = 16
SEQ_LEN = 200
BATCH = 4096

NC = 2
NS = 16
NW = NC * NS

ST = SEQ_LEN // 8
BT = BATCH // 128
N_BLOCKS = ST * BT
BLK_PER_W = N_BLOCKS // NW
ROWS_BLK = 8 * 128
OB_LEN = ROWS_BLK * EMBED_DIM


def _positional_encoding_host(seq_len, embed_dim):
    pos = jnp.arange(seq_len, dtype=jnp.float32)[:, None]
    dim = jnp.arange(embed_dim, dtype=jnp.float32)[None, :]
    angle = pos / jnp.power(10000.0, 2.0 * dim / float(embed_dim))
    is_even = (jnp.arange(embed_dim)[None, :] % 2) == 0
    return jnp.where(is_even, jnp.sin(angle), jnp.cos(angle))


def _sc_body(table_hbm, xr_hbm, pe_hbm, out_hbm,
             idx_all, rows0, rows1, ob0, ob1, pe_v,
             gsem0, gsem1, osem0, osem1):
    w = lax.axis_index("s") * NC + lax.axis_index("c")
    rows_b = (rows0, rows1)
    ob_b = (ob0, ob1)
    gsem_b = (gsem0, gsem1)
    osem_b = (osem0, osem1)

    pltpu.sync_copy(xr_hbm.at[pl.ds(w * BLK_PER_W, BLK_PER_W)], idx_all)
    pltpu.sync_copy(pe_hbm, pe_v)
    lane = lax.iota(jnp.int32, 16)

    def tiles(k):
        blk = w * BLK_PER_W + k
        return blk // BT, blk % BT

    def gather_start(k, b):
        pltpu.async_copy(table_hbm.at[idx_all.at[k]], rows_b[b], gsem_b[b])

    def gather_wait(k, b):
        pltpu.make_async_copy(table_hbm.at[idx_all.at[k]], rows_b[b],
                              gsem_b[b]).wait()

    def store_start(k, b):
        st, bt = tiles(k)
        obuf = ob_b[b]
        for sr in range(8):
            for dt in range(2):
                pltpu.async_copy(
                    obuf.at[pl.ds(sr * 16 + dt * 8, 8), pl.ds(0, 128)],
                    out_hbm.at[st * 8 + sr, dt, bt], osem_b[b])

    def store_wait(k, b):
        st, bt = tiles(k)
        obuf = ob_b[b]
        for sr in range(8):
            for dt in range(2):
                pltpu.make_async_copy(
                    obuf.at[pl.ds(sr * 16 + dt * 8, 8), pl.ds(0, 128)],
                    out_hbm.at[st * 8 + sr, dt, bt], osem_b[b]).wait()

    def transpose(k, b):
        st, _ = tiles(k)
        rows_v = rows_b[b]
        obuf = ob_b[b]

        def sr_step(sr, _):
            pe_row = pe_v[st * 8 + sr]
            rowv = lane + sr * 16
            for bl in range(128):
                vals = rows_v[sr * 128 + bl] + pe_row
                plsc.store_scatter(obuf, [rowv, jnp.full((16,), bl,
                                                         jnp.int32)], vals)
            return 0

        lax.fori_loop(0, 8, sr_step, 0)

    gather_start(0, 0)

    def pair_step(kk, carry):
        for par in range(2):
            k = 2 * kk + par
            b = par
            gather_wait(k, b)
            gather_start(k + 1, 1 - b)

            @pl.when(k >= 2)
            def _():
                store_wait(k - 2, b)

            transpose(k, b)
            store_start(k, b)
        return carry

    lax.fori_loop(0, (BLK_PER_W - 1) // 2, pair_step, 0)

    last = BLK_PER_W - 1
    gather_wait(last, 0)
    store_wait(last - 2, 0)
    transpose(last, 0)
    store_start(last, 0)
    store_wait(last - 1, 1)
    store_wait(last, 0)


@jax.jit
def _embed_lookup(xr, table, pe):
    mesh = plsc.VectorSubcoreMesh(core_axis_name="c", subcore_axis_name="s")
    return pl.kernel(
        _sc_body,
        out_type=jax.ShapeDtypeStruct((SEQ_LEN, 2, BT, 8, 128),
                                      jnp.float32),
        mesh=mesh,
        scratch_types=[
            pltpu.VMEM((N_BLOCKS // NW, ROWS_BLK), jnp.int32),
            pltpu.VMEM((ROWS_BLK, EMBED_DIM), jnp.float32),
            pltpu.VMEM((ROWS_BLK, EMBED_DIM), jnp.float32),
            pltpu.VMEM((128, 129), jnp.float32),
            pltpu.VMEM((128, 129), jnp.float32),
            pltpu.VMEM((SEQ_LEN, EMBED_DIM), jnp.float32),
            pltpu.SemaphoreType.DMA,
            pltpu.SemaphoreType.DMA,
            pltpu.SemaphoreType.DMA,
            pltpu.SemaphoreType.DMA,
        ],
        compiler_params=pltpu.CompilerParams(use_tc_tiling_on_sc=False,
                                             needs_layout_passes=False),
    )(table, xr, pe)


def kernel(x, table):
    pe = _positional_encoding_host(SEQ_LEN, EMBED_DIM)
    xr = (x.astype(jnp.int32).T
          .reshape(ST, 8, BT, 128)
          .transpose(0, 2, 1, 3)
          .reshape(N_BLOCKS, ROWS_BLK))
    out5 = _embed_lookup(xr, table, pe)
    return (out5.transpose(2, 4, 0, 1, 3)
            .reshape(BATCH, SEQ_LEN, EMBED_DIM))
